# baseline (speedup 1.0000x reference)
import functools
import jax
import jax.numpy as jnp
from jax.experimental import pallas as pl
from jax.experimental.pallas import tpu as pltpu

B, N, E, H, EMB = 16, 10000, 160000, 128, 256
NT = B * N


def _mm(a, b):
    return jax.lax.dot_general(a, b, (((1,), (0,)), ((), ())),
                               precision=jax.lax.Precision.HIGHEST,
                               preferred_element_type=jnp.float32)


def _head_body(agg_ref, act_ref, step_ref, W2_ref, b2_ref, P1m_ref, P1a_ref,
               P1s_ref, P1b_ref, P2w_ref, P2b_ref, out_ref):
    acc = (_mm(act_ref[...], P1a_ref[...].T) + _mm(step_ref[...], P1s_ref[...].T)
           + P1b_ref[...][None, :])
    W2t = W2_ref[...].T
    b2 = b2_ref[...][None, :]
    for j in range(5):
        o2 = jnp.maximum(_mm(agg_ref[j], W2t) + b2, 0.0)
        acc = acc + _mm(o2, P1m_ref[j])
    h = jnp.maximum(acc, 0.0)
    out_ref[...] = _mm(h, P2w_ref[...].T) + P2b_ref[...][None, :]


def _head(agg, actions, steps, W2, b2, P1m, P1a, P1s, P1b, P2w, P2b):
    return pl.pallas_call(
        _head_body,
        out_shape=jax.ShapeDtypeStruct((B, EMB), jnp.float32),
    )(agg, actions, steps, W2, b2, P1m, P1a, P1s, P1b, P2w, P2b)


def kernel(graph_x, graph_edges, position_seq, available_moves, actions, steps,
           W1, b1, W2, b2, P1w, P1b, P2w, P2b):
    addition = jnp.arange(B, dtype=graph_edges.dtype) * N
    x = graph_x.reshape(-1, graph_x.shape[2])
    ge = graph_edges + addition.reshape(-1, 1, 1)
    ge = jnp.transpose(ge, (1, 0, 2)).reshape(2, -1)
    r, c = ge[0], ge[1]
    targets = jnp.concatenate(
        [available_moves + addition[:, None], (position_seq + addition)[:, None]],
        axis=1)

    deg = jnp.ones((NT,), jnp.float32).at[c].add(1.0)
    dis = jax.lax.rsqrt(deg)
    y = dis[:, None] * x
    S = jnp.zeros((NT, 5), jnp.float32).at[c].add(y[r])
    z1 = dis[:, None] * (S + y)

    tflat = targets.reshape(-1)
    lut = jnp.full((NT,), -1, jnp.int32).at[tflat].set(
        jnp.arange(80, dtype=jnp.int32))
    sel = lut[c]
    K = 4096
    (eidx,) = jnp.nonzero(sel >= 0, size=K, fill_value=c.shape[0])
    valid = eidx < c.shape[0]
    eidx_c = jnp.minimum(eidx, c.shape[0] - 1)
    rsub = r[eidx_c]
    slot = jnp.where(valid, sel[eidx_c], 80)
    w = jnp.where(valid, dis[rsub], 0.0)
    x2sub = jnp.maximum(z1[rsub] @ W1.T + b1, 0.0)
    aggW = jnp.zeros((81, H), jnp.float32).at[slot].add(w[:, None] * x2sub)
    x2t = jnp.maximum(z1[tflat] @ W1.T + b1, 0.0)
    winner = lut[tflat]
    agg = aggW[winner] + dis[tflat][:, None] * x2t
    agg = dis[tflat][:, None] * agg
    agg = agg.reshape(B, 5, H).transpose(1, 0, 2)

    P1m = P1w[:, :5 * H].reshape(EMB, 5, H).transpose(1, 2, 0)
    P1a = P1w[:, 5 * H:5 * H + 8]
    P1s = P1w[:, 5 * H + 8:]
    out = _head(agg, actions, steps, W2, b2, P1m, P1a, P1s, P1b, P2w, P2b)
    return out


# speedup vs baseline: 2.8104x; 2.8104x over previous
import functools
import jax
import jax.numpy as jnp
from jax import lax
from jax.experimental import pallas as pl
from jax.experimental.pallas import tpu as pltpu
from jax.experimental.pallas import tpu_sc as plsc

B, N, E, H, EMB = 16, 10000, 160000, 128, 256
NT = B * N
BE = B * E
NC, NS = 2, 16
NW = NC * NS
ROWS = BE // 128
RPT = ROWS // NW
SLICE = NT // NS

_mesh = plsc.VectorSubcoreMesh(core_axis_name="c", subcore_axis_name="s")


def _deg_body(cols_hbm, zeros_hbm, out_hbm, deg_sh, colbuf, ones_v, sem):
    cid = lax.axis_index("c")
    sid = lax.axis_index("s")
    wid = cid * NS + sid
    for i in range(8):
        ones_v[pl.ds(i * 16, 16)] = jnp.ones((16,), jnp.float32)
    pltpu.sync_copy(zeros_hbm.at[pl.ds(sid * SLICE, SLICE)],
                    deg_sh.at[pl.ds(sid * SLICE, SLICE)])
    plsc.subcore_barrier()
    base = wid * RPT

    def step(i, carry):
        pltpu.sync_copy(cols_hbm.at[pl.ds(base + i * 5, 5)], colbuf)
        descs = [pltpu.async_copy(ones_v, deg_sh.at[colbuf.at[j]], sem,
                                  add=True) for j in range(5)]
        for d in descs:
            d.wait()
        return carry

    lax.fori_loop(0, RPT // 5, step, 0)
    plsc.subcore_barrier()
    pltpu.sync_copy(deg_sh.at[pl.ds(sid * SLICE, SLICE)],
                    out_hbm.at[cid, pl.ds(sid * SLICE, SLICE)])


_deg_kernel = pl.kernel(
    _deg_body,
    out_type=jax.ShapeDtypeStruct((NC, NT), jnp.float32),
    mesh=_mesh,
    compiler_params=pltpu.CompilerParams(use_tc_tiling_on_sc=False),
    scratch_types=[
        pltpu.VMEM_SHARED((NT,), jnp.float32),
        pltpu.VMEM((5, 128), jnp.int32),
        pltpu.VMEM((128,), jnp.float32),
        pltpu.SemaphoreType.DMA,
    ],
)


def _agg_body(rows_hbm, cols_hbm, y_hbm, zeros_hbm, out_hbm,
              S_sh, rbuf, cbuf, ybuf, sem, sem2):
    cid = lax.axis_index("c")
    sid = lax.axis_index("s")
    wid = cid * NS + sid
    pltpu.sync_copy(zeros_hbm.at[pl.ds(sid * SLICE, SLICE)],
                    S_sh.at[pl.ds(sid * SLICE, SLICE)])
    plsc.subcore_barrier()
    base = wid * RPT

    def step(i, carry):
        pltpu.sync_copy(rows_hbm.at[pl.ds(base + i * 5, 5)], rbuf)
        pltpu.sync_copy(cols_hbm.at[pl.ds(base + i * 5, 5)], cbuf)
        g = [pltpu.async_copy(y_hbm.at[rbuf.at[j]], ybuf.at[j], sem)
             for j in range(5)]
        for d in g:
            d.wait()
        s = [pltpu.async_copy(ybuf.at[j], S_sh.at[cbuf.at[j]], sem2, add=True)
             for j in range(5)]
        for d in s:
            d.wait()
        return carry

    lax.fori_loop(0, RPT // 5, step, 0)
    plsc.subcore_barrier()
    pltpu.sync_copy(S_sh.at[pl.ds(sid * SLICE, SLICE)],
                    out_hbm.at[cid, pl.ds(sid * SLICE, SLICE)])


_agg_kernel = pl.kernel(
    _agg_body,
    out_type=jax.ShapeDtypeStruct((NC, NT, 8), jnp.float32),
    mesh=_mesh,
    compiler_params=pltpu.CompilerParams(use_tc_tiling_on_sc=False),
    scratch_types=[
        pltpu.VMEM_SHARED((NT, 8), jnp.float32),
        pltpu.VMEM((5, 128), jnp.int32),
        pltpu.VMEM((5, 128), jnp.int32),
        pltpu.VMEM((5, 128, 8), jnp.float32),
        pltpu.SemaphoreType.DMA,
        pltpu.SemaphoreType.DMA,
    ],
)


def _mm(a, b):
    return jax.lax.dot_general(a, b, (((1,), (0,)), ((), ())),
                               precision=jax.lax.Precision.HIGHEST,
                               preferred_element_type=jnp.float32)


def _head_body(agg_ref, act_ref, step_ref, W2_ref, b2_ref, P1m_ref, P1a_ref,
               P1s_ref, P1b_ref, P2w_ref, P2b_ref, out_ref):
    acc = (_mm(act_ref[...], P1a_ref[...].T) + _mm(step_ref[...], P1s_ref[...].T)
           + P1b_ref[...][None, :])
    W2t = W2_ref[...].T
    b2 = b2_ref[...][None, :]
    for j in range(5):
        o2 = jnp.maximum(_mm(agg_ref[j], W2t) + b2, 0.0)
        acc = acc + _mm(o2, P1m_ref[j])
    h = jnp.maximum(acc, 0.0)
    out_ref[...] = _mm(h, P2w_ref[...].T) + P2b_ref[...][None, :]


def _head(agg, actions, steps, W2, b2, P1m, P1a, P1s, P1b, P2w, P2b):
    return pl.pallas_call(
        _head_body,
        out_shape=jax.ShapeDtypeStruct((B, EMB), jnp.float32),
    )(agg, actions, steps, W2, b2, P1m, P1a, P1s, P1b, P2w, P2b)


def kernel(graph_x, graph_edges, position_seq, available_moves, actions, steps,
           W1, b1, W2, b2, P1w, P1b, P2w, P2b):
    addition = jnp.arange(B, dtype=graph_edges.dtype) * N
    x = graph_x.reshape(-1, graph_x.shape[2])
    ge = graph_edges + addition.reshape(-1, 1, 1)
    ge = jnp.transpose(ge, (1, 0, 2)).reshape(2, -1)
    r, c = ge[0], ge[1]
    r2d = r.reshape(ROWS, 128)
    c2d = c.reshape(ROWS, 128)
    targets = jnp.concatenate(
        [available_moves + addition[:, None], (position_seq + addition)[:, None]],
        axis=1)

    degp = _deg_kernel(c2d, jnp.zeros((NT,), jnp.float32))
    deg = 1.0 + degp[0] + degp[1]
    dis = jax.lax.rsqrt(deg)
    x8 = jnp.pad(x, ((0, 0), (0, 3)))
    y8 = dis[:, None] * x8
    Sp = _agg_kernel(r2d, c2d, y8, jnp.zeros((NT, 8), jnp.float32))
    z18 = dis[:, None] * (Sp[0] + Sp[1] + y8)
    z1 = z18[:, :5]

    tflat = targets.reshape(-1)
    lut = jnp.full((NT,), -1, jnp.int32).at[tflat].set(
        jnp.arange(80, dtype=jnp.int32))
    sel = lut[c]
    K = 4096
    (eidx,) = jnp.nonzero(sel >= 0, size=K, fill_value=c.shape[0])
    valid = eidx < c.shape[0]
    eidx_c = jnp.minimum(eidx, c.shape[0] - 1)
    rsub = r[eidx_c]
    slot = jnp.where(valid, sel[eidx_c], 80)
    w = jnp.where(valid, dis[rsub], 0.0)
    x2sub = jnp.maximum(z1[rsub] @ W1.T + b1, 0.0)
    aggW = jnp.zeros((81, H), jnp.float32).at[slot].add(w[:, None] * x2sub)
    x2t = jnp.maximum(z1[tflat] @ W1.T + b1, 0.0)
    winner = lut[tflat]
    agg = aggW[winner] + dis[tflat][:, None] * x2t
    agg = dis[tflat][:, None] * agg
    agg = agg.reshape(B, 5, H).transpose(1, 0, 2)

    P1m = P1w[:, :5 * H].reshape(EMB, 5, H).transpose(1, 2, 0)
    P1a = P1w[:, 5 * H:5 * H + 8]
    P1s = P1w[:, 5 * H + 8:]
    out = _head(agg, actions, steps, W2, b2, P1m, P1a, P1s, P1b, P2w, P2b)
    return out


# speedup vs baseline: 53.4252x; 19.0096x over previous
import functools
import jax
import jax.numpy as jnp
from jax import lax
from jax.experimental import pallas as pl
from jax.experimental.pallas import tpu as pltpu
from jax.experimental.pallas import tpu_sc as plsc

B, N, E, H, EMB = 16, 10000, 160000, 128, 256
NT = B * N
BE = B * E
NC, NS = 2, 16
NW = NC * NS
ROWS = BE // 128
RPT = ROWS // NW
SLICE = NT // NS

_mesh = plsc.VectorSubcoreMesh(core_axis_name="c", subcore_axis_name="s")


def _deg_body(cols_hbm, zeros_hbm, out_hbm, deg_sh, colbuf, ones_v, sem):
    cid = lax.axis_index("c")
    sid = lax.axis_index("s")
    wid = cid * NS + sid
    for i in range(8):
        ones_v[pl.ds(i * 16, 16)] = jnp.ones((16,), jnp.float32)
    pltpu.sync_copy(zeros_hbm.at[pl.ds(sid * SLICE, SLICE)],
                    deg_sh.at[pl.ds(sid * SLICE, SLICE)])
    plsc.subcore_barrier()
    base = wid * RPT

    def step(i, carry):
        pltpu.sync_copy(cols_hbm.at[pl.ds(base + i * 5, 5)], colbuf)
        descs = [pltpu.async_copy(ones_v, deg_sh.at[colbuf.at[j]], sem,
                                  add=True) for j in range(5)]
        for d in descs:
            d.wait()
        return carry

    lax.fori_loop(0, RPT // 5, step, 0)
    plsc.subcore_barrier()
    pltpu.sync_copy(deg_sh.at[pl.ds(sid * SLICE, SLICE)],
                    out_hbm.at[cid, pl.ds(sid * SLICE, SLICE)])


_deg_kernel = pl.kernel(
    _deg_body,
    out_type=jax.ShapeDtypeStruct((NC, NT), jnp.float32),
    mesh=_mesh,
    compiler_params=pltpu.CompilerParams(use_tc_tiling_on_sc=False),
    scratch_types=[
        pltpu.VMEM_SHARED((NT,), jnp.float32),
        pltpu.VMEM((5, 128), jnp.int32),
        pltpu.VMEM((128,), jnp.float32),
        pltpu.SemaphoreType.DMA,
    ],
)


def _agg_body(rows_hbm, cols_hbm, y_hbm, zeros_hbm, out_hbm,
              S_sh, rbuf, cbuf, ybuf, sem, sem2):
    cid = lax.axis_index("c")
    sid = lax.axis_index("s")
    wid = cid * NS + sid
    pltpu.sync_copy(zeros_hbm.at[pl.ds(sid * SLICE, SLICE)],
                    S_sh.at[pl.ds(sid * SLICE, SLICE)])
    plsc.subcore_barrier()
    base = wid * RPT

    def step(i, carry):
        pltpu.sync_copy(rows_hbm.at[pl.ds(base + i * 5, 5)], rbuf)
        pltpu.sync_copy(cols_hbm.at[pl.ds(base + i * 5, 5)], cbuf)
        g = [pltpu.async_copy(y_hbm.at[rbuf.at[j]], ybuf.at[j], sem)
             for j in range(5)]
        for d in g:
            d.wait()
        s = [pltpu.async_copy(ybuf.at[j], S_sh.at[cbuf.at[j]], sem2, add=True)
             for j in range(5)]
        for d in s:
            d.wait()
        return carry

    lax.fori_loop(0, RPT // 5, step, 0)
    plsc.subcore_barrier()
    pltpu.sync_copy(S_sh.at[pl.ds(sid * SLICE, SLICE)],
                    out_hbm.at[cid, pl.ds(sid * SLICE, SLICE)])


_agg_kernel = pl.kernel(
    _agg_body,
    out_type=jax.ShapeDtypeStruct((NC, NT, 8), jnp.float32),
    mesh=_mesh,
    compiler_params=pltpu.CompilerParams(use_tc_tiling_on_sc=False),
    scratch_types=[
        pltpu.VMEM_SHARED((NT, 8), jnp.float32),
        pltpu.VMEM((5, 128), jnp.int32),
        pltpu.VMEM((5, 128), jnp.int32),
        pltpu.VMEM((5, 128, 8), jnp.float32),
        pltpu.SemaphoreType.DMA,
        pltpu.SemaphoreType.DMA,
    ],
)


EPT = BE // NW
NCHUNK = 25
CSZ = EPT // NCHUNK
NVEC = CSZ // 16
LCAP = CSZ + 32


def _match_body(c_hbm, r_hbm, z1t_hbm, tgt_hbm, w1p_hbm, out_hbm,
                cbuf, rbuf, zbuf, wbuf, tb, accv, sem):
    cid = lax.axis_index("c")
    sid = lax.axis_index("s")
    wid = cid * NS + sid
    parity = lax.rem(wid, 2)
    pltpu.sync_copy(w1p_hbm, wbuf)
    pltpu.sync_copy(tgt_hbm.at[wid], tb)
    tvec = tb[...]
    tsc = [tvec[j] for j in range(5)]
    zv = jnp.zeros((16,), jnp.float32)
    for j in range(5):
        for cc in range(8):
            accv[j, cc] = zv
    ebase = wid * EPT

    def chunk_step(ci, carry0):
        pltpu.sync_copy(c_hbm.at[pl.ds(ebase + ci * CSZ, CSZ)], cbuf)
        pltpu.sync_copy(r_hbm.at[pl.ds(ebase + ci * CSZ, CSZ)], rbuf)

        def vec_step(v, carry):
            cv = cbuf[pl.ds(v * 16, 16)]
            m = [cv == tsc[j] for j in range(5)]
            many = m[0]
            for j in range(1, 5):
                many = jnp.logical_or(many, m[j])
            hb = jnp.where(many, 1, 0)
            s = hb[0]
            for q in range(1, 16):
                s = s | hb[q]

            @pl.when(s > 0)
            def _():
                rv = rbuf[pl.ds(v * 16, 16)]
                idxs = jnp.where(many, rv, NT)
                pltpu.async_copy(z1t_hbm.at[idxs], zbuf, sem).wait()
                mw = [jnp.where(m[j], 1.0, 0.0) for j in range(5)]
                for i in range(16):
                    zr = zbuf[i]
                    w = zr[5]
                    wj = [w * mw[j][i] for j in range(5)]
                    for cc in range(8):
                        t = wbuf[5, pl.ds(cc * 16, 16)]
                        for k in range(5):
                            t = t + zr[k] * wbuf[k, pl.ds(cc * 16, 16)]
                        x2 = jnp.maximum(t, 0.0)
                        for j in range(5):
                            accv[j, cc] = accv[j, cc] + wj[j] * x2
            return carry

        lax.fori_loop(0, NVEC, vec_step, 0)
        return carry0

    lax.fori_loop(0, NCHUNK, chunk_step, 0)
    pltpu.async_copy(z1t_hbm.at[tvec], zbuf, sem).wait()

    @pl.when(parity == 0)
    def _():
        for j in range(5):
            zr = zbuf[j]
            w = zr[5]
            for cc in range(8):
                t = wbuf[5, pl.ds(cc * 16, 16)]
                for k in range(5):
                    t = t + zr[k] * wbuf[k, pl.ds(cc * 16, 16)]
                accv[j, cc] = accv[j, cc] + w * jnp.maximum(t, 0.0)

    for j in range(5):
        dp = zbuf[j][5]
        for cc in range(8):
            accv[j, cc] = accv[j, cc] * dp
    pltpu.sync_copy(accv, out_hbm.at[wid])


_match_kernel = pl.kernel(
    _match_body,
    out_type=jax.ShapeDtypeStruct((NW, 5, 8, 16), jnp.float32),
    mesh=_mesh,
    compiler_params=pltpu.CompilerParams(use_tc_tiling_on_sc=False),
    scratch_types=[
        pltpu.VMEM((CSZ,), jnp.int32),
        pltpu.VMEM((CSZ,), jnp.int32),
        pltpu.VMEM((16, 16), jnp.float32),
        pltpu.VMEM((8, 128), jnp.float32),
        pltpu.VMEM((16,), jnp.int32),
        pltpu.VMEM((5, 8, 16), jnp.float32),
        pltpu.SemaphoreType.DMA,
    ],
)


def _mm(a, b):
    return jax.lax.dot_general(a, b, (((1,), (0,)), ((), ())),
                               precision=jax.lax.Precision.HIGHEST,
                               preferred_element_type=jnp.float32)


def _head_body(agg_ref, act_ref, step_ref, W2_ref, b2_ref, P1m_ref, P1a_ref,
               P1s_ref, P1b_ref, P2w_ref, P2b_ref, out_ref):
    acc = (_mm(act_ref[...], P1a_ref[...].T) + _mm(step_ref[...], P1s_ref[...].T)
           + P1b_ref[...][None, :])
    W2t = W2_ref[...].T
    b2 = b2_ref[...][None, :]
    for j in range(5):
        o2 = jnp.maximum(_mm(agg_ref[j], W2t) + b2, 0.0)
        acc = acc + _mm(o2, P1m_ref[j])
    h = jnp.maximum(acc, 0.0)
    out_ref[...] = _mm(h, P2w_ref[...].T) + P2b_ref[...][None, :]


def _head(agg, actions, steps, W2, b2, P1m, P1a, P1s, P1b, P2w, P2b):
    return pl.pallas_call(
        _head_body,
        out_shape=jax.ShapeDtypeStruct((B, EMB), jnp.float32),
    )(agg, actions, steps, W2, b2, P1m, P1a, P1s, P1b, P2w, P2b)


def kernel(graph_x, graph_edges, position_seq, available_moves, actions, steps,
           W1, b1, W2, b2, P1w, P1b, P2w, P2b):
    addition = jnp.arange(B, dtype=graph_edges.dtype) * N
    x = graph_x.reshape(-1, graph_x.shape[2])
    ge = graph_edges + addition.reshape(-1, 1, 1)
    ge = jnp.transpose(ge, (1, 0, 2)).reshape(2, -1)
    r, c = ge[0], ge[1]
    r2d = r.reshape(ROWS, 128)
    c2d = c.reshape(ROWS, 128)
    targets = jnp.concatenate(
        [available_moves + addition[:, None], (position_seq + addition)[:, None]],
        axis=1)

    degp = _deg_kernel(c2d, jnp.zeros((NT,), jnp.float32))
    deg = 1.0 + degp[0] + degp[1]
    dis = jax.lax.rsqrt(deg)
    x8 = jnp.pad(x, ((0, 0), (0, 3)))
    y8 = dis[:, None] * x8
    Sp = _agg_kernel(r2d, c2d, y8, jnp.zeros((NT, 8), jnp.float32))
    z18 = dis[:, None] * (Sp[0] + Sp[1] + y8)

    z1t = jnp.concatenate(
        [z18[:, :5], dis[:, None], jnp.zeros((NT, 10), jnp.float32)], axis=1)
    z1t = jnp.concatenate([z1t, jnp.zeros((16, 16), jnp.float32)], axis=0)
    tgt = jnp.repeat(targets, 2, axis=0)
    tgt = jnp.pad(tgt, ((0, 0), (0, 11)), constant_values=NT)
    W1p = jnp.concatenate(
        [W1.T, b1[None, :], jnp.zeros((2, H), jnp.float32)], axis=0)
    accp = _match_kernel(c, r, z1t, tgt, W1p)
    agg = accp.reshape(B, 2, 5, H).sum(axis=1)
    agg = agg.transpose(1, 0, 2)

    P1m = P1w[:, :5 * H].reshape(EMB, 5, H).transpose(1, 2, 0)
    P1a = P1w[:, 5 * H:5 * H + 8]
    P1s = P1w[:, 5 * H + 8:]
    out = _head(agg, actions, steps, W2, b2, P1m, P1a, P1s, P1b, P2w, P2b)
    return out


# speedup vs baseline: 53.9714x; 1.0102x over previous
import functools
import jax
import jax.numpy as jnp
from jax import lax
from jax.experimental import pallas as pl
from jax.experimental.pallas import tpu as pltpu
from jax.experimental.pallas import tpu_sc as plsc

B, N, E, H, EMB = 16, 10000, 160000, 128, 256
NT = B * N
BE = B * E
NC, NS = 2, 16
NW = NC * NS
ROWS = BE // 128
RPT = ROWS // NW
SLICE = NT // NS

_mesh = plsc.VectorSubcoreMesh(core_axis_name="c", subcore_axis_name="s")


def _deg_body(cols_hbm, zeros_hbm, out_hbm, deg_sh, colbuf, ones_v,
              sema, semb):
    cid = lax.axis_index("c")
    sid = lax.axis_index("s")
    wid = cid * NS + sid
    for i in range(8):
        ones_v[pl.ds(i * 16, 16)] = jnp.ones((16,), jnp.float32)
    pltpu.sync_copy(zeros_hbm.at[pl.ds(sid * SLICE, SLICE)],
                    deg_sh.at[pl.ds(sid * SLICE, SLICE)])
    plsc.subcore_barrier()
    base = wid * RPT
    GR = 25
    NG = RPT // GR

    def step(i, carry):
        par = lax.rem(i, 2)
        for p, sem in ((0, sema), (1, semb)):
            @pl.when(jnp.logical_and(par == p, i >= 2))
            def _():
                for j in range(GR):
                    pltpu.make_async_copy(
                        ones_v, deg_sh.at[colbuf.at[p, j]], sem).wait()

            @pl.when(jnp.logical_and(par == p, i < NG))
            def _():
                pltpu.sync_copy(cols_hbm.at[pl.ds(base + i * GR, GR)],
                                colbuf.at[p])
                for j in range(GR):
                    pltpu.async_copy(ones_v, deg_sh.at[colbuf.at[p, j]],
                                     sem, add=True)
        return carry

    lax.fori_loop(0, NG + 2, step, 0)
    plsc.subcore_barrier()
    pltpu.sync_copy(deg_sh.at[pl.ds(sid * SLICE, SLICE)],
                    out_hbm.at[cid, pl.ds(sid * SLICE, SLICE)])


_deg_kernel = pl.kernel(
    _deg_body,
    out_type=jax.ShapeDtypeStruct((NC, NT), jnp.float32),
    mesh=_mesh,
    compiler_params=pltpu.CompilerParams(use_tc_tiling_on_sc=False),
    scratch_types=[
        pltpu.VMEM_SHARED((NT,), jnp.float32),
        pltpu.VMEM((2, 25, 128), jnp.int32),
        pltpu.VMEM((128,), jnp.float32),
        pltpu.SemaphoreType.DMA,
        pltpu.SemaphoreType.DMA,
    ],
)


def _agg_body(rows_hbm, cols_hbm, y_hbm, zeros_hbm, out_hbm,
              S_sh, rbuf, cbuf, ybuf, gsa, gsb, ssa, ssb):
    cid = lax.axis_index("c")
    sid = lax.axis_index("s")
    wid = cid * NS + sid
    pltpu.sync_copy(zeros_hbm.at[pl.ds(sid * SLICE, SLICE)],
                    S_sh.at[pl.ds(sid * SLICE, SLICE)])
    plsc.subcore_barrier()
    base = wid * RPT
    GR = 5
    NG = RPT // GR

    def step(i, carry):
        par = lax.rem(i, 2)
        for p, gsem, ssem in ((0, gsa, ssa), (1, gsb, ssb)):
            @pl.when(jnp.logical_and(par == p, i >= 2))
            def _():
                for j in range(GR):
                    pltpu.make_async_copy(
                        ybuf.at[p, j], S_sh.at[cbuf.at[p, j]], ssem).wait()

            @pl.when(jnp.logical_and(par == p, i < NG))
            def _():
                pltpu.sync_copy(rows_hbm.at[pl.ds(base + i * GR, GR)],
                                rbuf.at[p])
                pltpu.sync_copy(cols_hbm.at[pl.ds(base + i * GR, GR)],
                                cbuf.at[p])
                for j in range(GR):
                    pltpu.async_copy(y_hbm.at[rbuf.at[p, j]], ybuf.at[p, j],
                                     gsem)
                for j in range(GR):
                    pltpu.make_async_copy(y_hbm.at[rbuf.at[p, j]],
                                          ybuf.at[p, j], gsem).wait()
                for j in range(GR):
                    pltpu.async_copy(ybuf.at[p, j], S_sh.at[cbuf.at[p, j]],
                                     ssem, add=True)
        return carry

    lax.fori_loop(0, NG + 2, step, 0)
    plsc.subcore_barrier()
    pltpu.sync_copy(S_sh.at[pl.ds(sid * SLICE, SLICE)],
                    out_hbm.at[cid, pl.ds(sid * SLICE, SLICE)])


_agg_kernel = pl.kernel(
    _agg_body,
    out_type=jax.ShapeDtypeStruct((NC, NT, 8), jnp.float32),
    mesh=_mesh,
    compiler_params=pltpu.CompilerParams(use_tc_tiling_on_sc=False),
    scratch_types=[
        pltpu.VMEM_SHARED((NT, 8), jnp.float32),
        pltpu.VMEM((2, 5, 128), jnp.int32),
        pltpu.VMEM((2, 5, 128), jnp.int32),
        pltpu.VMEM((2, 5, 128, 8), jnp.float32),
        pltpu.SemaphoreType.DMA,
        pltpu.SemaphoreType.DMA,
        pltpu.SemaphoreType.DMA,
        pltpu.SemaphoreType.DMA,
    ],
)


EPT = BE // NW
NCHUNK = 25
CSZ = EPT // NCHUNK
NVEC = CSZ // 16
LCAP = CSZ + 32


def _match_body(c_hbm, r_hbm, z1t_hbm, tgt_hbm, w1p_hbm, out_hbm,
                cbuf, rbuf, zbuf, wbuf, tb, accv, sem):
    cid = lax.axis_index("c")
    sid = lax.axis_index("s")
    wid = cid * NS + sid
    parity = lax.rem(wid, 2)
    pltpu.sync_copy(w1p_hbm, wbuf)
    pltpu.sync_copy(tgt_hbm.at[wid], tb)
    tvec = tb[...]
    tsc = [tvec[j] for j in range(5)]
    zv = jnp.zeros((16,), jnp.float32)
    for j in range(5):
        for cc in range(8):
            accv[j, cc] = zv
    ebase = wid * EPT

    def chunk_step(ci, carry0):
        pltpu.sync_copy(c_hbm.at[pl.ds(ebase + ci * CSZ, CSZ)], cbuf)
        pltpu.sync_copy(r_hbm.at[pl.ds(ebase + ci * CSZ, CSZ)], rbuf)

        def vec_step(v4, carry):
            manys = []
            for u in range(4):
                cv = cbuf[pl.ds((v4 * 4 + u) * 16, 16)]
                mu = cv == tsc[0]
                for j in range(1, 5):
                    mu = jnp.logical_or(mu, cv == tsc[j])
                manys.append(mu)
            comb = manys[0]
            for u in range(1, 4):
                comb = jnp.logical_or(comb, manys[u])
            hb = jnp.where(comb, 1, 0)
            s = hb[0]
            for q in range(1, 16):
                s = s | hb[q]

            @pl.when(s > 0)
            def _():
                for u in range(4):
                    v = v4 * 4 + u
                    cv = cbuf[pl.ds(v * 16, 16)]
                    m = [cv == tsc[j] for j in range(5)]
                    many = m[0]
                    for j in range(1, 5):
                        many = jnp.logical_or(many, m[j])
                    hbu = jnp.where(many, 1, 0)
                    su = hbu[0]
                    for q in range(1, 16):
                        su = su | hbu[q]

                    @pl.when(su > 0)
                    def _():
                        rv = rbuf[pl.ds(v * 16, 16)]
                        idxs = jnp.where(many, rv, NT)
                        pltpu.async_copy(z1t_hbm.at[idxs], zbuf, sem).wait()
                        mw = [jnp.where(m[j], 1.0, 0.0) for j in range(5)]
                        for i in range(16):
                            zr = zbuf[i]
                            w = zr[5]
                            wj = [w * mw[j][i] for j in range(5)]
                            for cc in range(8):
                                t = wbuf[5, pl.ds(cc * 16, 16)]
                                for k in range(5):
                                    t = t + zr[k] * wbuf[k, pl.ds(cc * 16, 16)]
                                x2 = jnp.maximum(t, 0.0)
                                for j in range(5):
                                    accv[j, cc] = accv[j, cc] + wj[j] * x2
            return carry

        lax.fori_loop(0, NVEC // 4, vec_step, 0)
        return carry0

    lax.fori_loop(0, NCHUNK, chunk_step, 0)
    pltpu.async_copy(z1t_hbm.at[tvec], zbuf, sem).wait()

    @pl.when(parity == 0)
    def _():
        for j in range(5):
            zr = zbuf[j]
            w = zr[5]
            for cc in range(8):
                t = wbuf[5, pl.ds(cc * 16, 16)]
                for k in range(5):
                    t = t + zr[k] * wbuf[k, pl.ds(cc * 16, 16)]
                accv[j, cc] = accv[j, cc] + w * jnp.maximum(t, 0.0)

    for j in range(5):
        dp = zbuf[j][5]
        for cc in range(8):
            accv[j, cc] = accv[j, cc] * dp
    pltpu.sync_copy(accv, out_hbm.at[wid])


_match_kernel = pl.kernel(
    _match_body,
    out_type=jax.ShapeDtypeStruct((NW, 5, 8, 16), jnp.float32),
    mesh=_mesh,
    compiler_params=pltpu.CompilerParams(use_tc_tiling_on_sc=False),
    scratch_types=[
        pltpu.VMEM((CSZ,), jnp.int32),
        pltpu.VMEM((CSZ,), jnp.int32),
        pltpu.VMEM((16, 16), jnp.float32),
        pltpu.VMEM((8, 128), jnp.float32),
        pltpu.VMEM((16,), jnp.int32),
        pltpu.VMEM((5, 8, 16), jnp.float32),
        pltpu.SemaphoreType.DMA,
    ],
)


def _mm(a, b):
    return jax.lax.dot_general(a, b, (((1,), (0,)), ((), ())),
                               precision=jax.lax.Precision.HIGHEST,
                               preferred_element_type=jnp.float32)


def _head_body(agg_ref, act_ref, step_ref, W2_ref, b2_ref, P1m_ref, P1a_ref,
               P1s_ref, P1b_ref, P2w_ref, P2b_ref, out_ref):
    acc = (_mm(act_ref[...], P1a_ref[...].T) + _mm(step_ref[...], P1s_ref[...].T)
           + P1b_ref[...][None, :])
    W2t = W2_ref[...].T
    b2 = b2_ref[...][None, :]
    for j in range(5):
        o2 = jnp.maximum(_mm(agg_ref[j], W2t) + b2, 0.0)
        acc = acc + _mm(o2, P1m_ref[j])
    h = jnp.maximum(acc, 0.0)
    out_ref[...] = _mm(h, P2w_ref[...].T) + P2b_ref[...][None, :]


def _head(agg, actions, steps, W2, b2, P1m, P1a, P1s, P1b, P2w, P2b):
    return pl.pallas_call(
        _head_body,
        out_shape=jax.ShapeDtypeStruct((B, EMB), jnp.float32),
    )(agg, actions, steps, W2, b2, P1m, P1a, P1s, P1b, P2w, P2b)


def kernel(graph_x, graph_edges, position_seq, available_moves, actions, steps,
           W1, b1, W2, b2, P1w, P1b, P2w, P2b):
    addition = jnp.arange(B, dtype=graph_edges.dtype) * N
    x = graph_x.reshape(-1, graph_x.shape[2])
    ge = graph_edges + addition.reshape(-1, 1, 1)
    ge = jnp.transpose(ge, (1, 0, 2)).reshape(2, -1)
    r, c = ge[0], ge[1]
    r2d = r.reshape(ROWS, 128)
    c2d = c.reshape(ROWS, 128)
    targets = jnp.concatenate(
        [available_moves + addition[:, None], (position_seq + addition)[:, None]],
        axis=1)

    degp = _deg_kernel(c2d, jnp.zeros((NT,), jnp.float32))
    deg = 1.0 + degp[0] + degp[1]
    dis = jax.lax.rsqrt(deg)
    x8 = jnp.pad(x, ((0, 0), (0, 3)))
    y8 = dis[:, None] * x8
    Sp = _agg_kernel(r2d, c2d, y8, jnp.zeros((NT, 8), jnp.float32))
    z18 = dis[:, None] * (Sp[0] + Sp[1] + y8)

    z1t = jnp.concatenate(
        [z18[:, :5], dis[:, None], jnp.zeros((NT, 10), jnp.float32)], axis=1)
    z1t = jnp.concatenate([z1t, jnp.zeros((16, 16), jnp.float32)], axis=0)
    tgt = jnp.repeat(targets, 2, axis=0)
    tgt = jnp.pad(tgt, ((0, 0), (0, 11)), constant_values=NT)
    W1p = jnp.concatenate(
        [W1.T, b1[None, :], jnp.zeros((2, H), jnp.float32)], axis=0)
    accp = _match_kernel(c, r, z1t, tgt, W1p)
    agg = accp.reshape(B, 2, 5, H).sum(axis=1)
    agg = agg.transpose(1, 0, 2)

    P1m = P1w[:, :5 * H].reshape(EMB, 5, H).transpose(1, 2, 0)
    P1a = P1w[:, 5 * H:5 * H + 8]
    P1s = P1w[:, 5 * H + 8:]
    out = _head(agg, actions, steps, W2, b2, P1m, P1a, P1s, P1b, P2w, P2b)
    return out


# speedup vs baseline: 57.3528x; 1.0627x over previous
import functools
import jax
import jax.numpy as jnp
from jax import lax
from jax.experimental import pallas as pl
from jax.experimental.pallas import tpu as pltpu
from jax.experimental.pallas import tpu_sc as plsc

B, N, E, H, EMB = 16, 10000, 160000, 128, 256
NT = B * N
BE = B * E
NC, NS = 2, 16
NW = NC * NS
ROWS = BE // 128
RPT = ROWS // NW
SLICE = NT // NS

_mesh = plsc.VectorSubcoreMesh(core_axis_name="c", subcore_axis_name="s")


def _deg_body(cols_hbm, zeros_hbm, out_hbm, deg_sh, colbuf, ones_v,
              sema, semb):
    cid = lax.axis_index("c")
    sid = lax.axis_index("s")
    wid = cid * NS + sid
    for i in range(8):
        ones_v[pl.ds(i * 16, 16)] = jnp.ones((16,), jnp.float32)
    pltpu.sync_copy(zeros_hbm.at[pl.ds(sid * SLICE, SLICE)],
                    deg_sh.at[pl.ds(sid * SLICE, SLICE)])
    plsc.subcore_barrier()
    base = wid * RPT
    GR = 25
    NG = RPT // GR

    def step(i, carry):
        par = lax.rem(i, 2)
        for p, sem in ((0, sema), (1, semb)):
            @pl.when(jnp.logical_and(par == p, i >= 2))
            def _():
                for j in range(GR):
                    pltpu.make_async_copy(
                        ones_v, deg_sh.at[colbuf.at[p, j]], sem).wait()

            @pl.when(jnp.logical_and(par == p, i < NG))
            def _():
                pltpu.sync_copy(cols_hbm.at[pl.ds(base + i * GR, GR)],
                                colbuf.at[p])
                for j in range(GR):
                    pltpu.async_copy(ones_v, deg_sh.at[colbuf.at[p, j]],
                                     sem, add=True)
        return carry

    lax.fori_loop(0, NG + 2, step, 0)
    plsc.subcore_barrier()
    pltpu.sync_copy(deg_sh.at[pl.ds(sid * SLICE, SLICE)],
                    out_hbm.at[cid, pl.ds(sid * SLICE, SLICE)])


_deg_kernel = pl.kernel(
    _deg_body,
    out_type=jax.ShapeDtypeStruct((NC, NT), jnp.float32),
    mesh=_mesh,
    compiler_params=pltpu.CompilerParams(use_tc_tiling_on_sc=False),
    scratch_types=[
        pltpu.VMEM_SHARED((NT,), jnp.float32),
        pltpu.VMEM((2, 25, 128), jnp.int32),
        pltpu.VMEM((128,), jnp.float32),
        pltpu.SemaphoreType.DMA,
        pltpu.SemaphoreType.DMA,
    ],
)


def _agg_body(rows_hbm, cols_hbm, y_hbm, zeros_hbm, out_hbm,
              S_sh, rbuf, cbuf, ybuf, gsa, gsb, ssa, ssb):
    cid = lax.axis_index("c")
    sid = lax.axis_index("s")
    wid = cid * NS + sid
    pltpu.sync_copy(zeros_hbm.at[pl.ds(sid * SLICE, SLICE)],
                    S_sh.at[pl.ds(sid * SLICE, SLICE)])
    plsc.subcore_barrier()
    base = wid * RPT
    GR = 5
    NG = RPT // GR

    def step(i, carry):
        par = lax.rem(i, 2)
        for p, gsem, ssem in ((0, gsa, ssa), (1, gsb, ssb)):
            @pl.when(jnp.logical_and(par == p, i >= 2))
            def _():
                for j in range(GR):
                    pltpu.make_async_copy(
                        ybuf.at[p, j], S_sh.at[cbuf.at[p, j]], ssem).wait()

            @pl.when(jnp.logical_and(par == p, i < NG))
            def _():
                pltpu.sync_copy(rows_hbm.at[pl.ds(base + i * GR, GR)],
                                rbuf.at[p])
                pltpu.sync_copy(cols_hbm.at[pl.ds(base + i * GR, GR)],
                                cbuf.at[p])
                for j in range(GR):
                    pltpu.async_copy(y_hbm.at[rbuf.at[p, j]], ybuf.at[p, j],
                                     gsem)
                for j in range(GR):
                    pltpu.make_async_copy(y_hbm.at[rbuf.at[p, j]],
                                          ybuf.at[p, j], gsem).wait()
                for j in range(GR):
                    pltpu.async_copy(ybuf.at[p, j], S_sh.at[cbuf.at[p, j]],
                                     ssem, add=True)
        return carry

    lax.fori_loop(0, NG + 2, step, 0)
    plsc.subcore_barrier()
    pltpu.sync_copy(S_sh.at[pl.ds(sid * SLICE, SLICE)],
                    out_hbm.at[cid, pl.ds(sid * SLICE, SLICE)])


_agg_kernel = pl.kernel(
    _agg_body,
    out_type=jax.ShapeDtypeStruct((NC, NT, 8), jnp.float32),
    mesh=_mesh,
    compiler_params=pltpu.CompilerParams(use_tc_tiling_on_sc=False),
    scratch_types=[
        pltpu.VMEM_SHARED((NT, 8), jnp.float32),
        pltpu.VMEM((2, 5, 128), jnp.int32),
        pltpu.VMEM((2, 5, 128), jnp.int32),
        pltpu.VMEM((2, 5, 128, 8), jnp.float32),
        pltpu.SemaphoreType.DMA,
        pltpu.SemaphoreType.DMA,
        pltpu.SemaphoreType.DMA,
        pltpu.SemaphoreType.DMA,
    ],
)


EPT = BE // NW
NCHUNK = 25
CSZ = EPT // NCHUNK
NVEC = CSZ // 16
LCAP = CSZ + 32


def _match_body(c_hbm, r_hbm, z1t_hbm, tgt_hbm, w1p_hbm, out_hbm,
                cbuf, rbuf, zbuf, wbuf, tb, accv, sem):
    cid = lax.axis_index("c")
    sid = lax.axis_index("s")
    wid = cid * NS + sid
    parity = lax.rem(wid, 2)
    pltpu.sync_copy(w1p_hbm, wbuf)
    pltpu.sync_copy(tgt_hbm.at[wid], tb)
    tvec = tb[...]
    tsc = [tvec[j] for j in range(5)]
    zv = jnp.zeros((16,), jnp.float32)
    for j in range(5):
        for cc in range(8):
            accv[j, cc] = zv
    ebase = wid * EPT

    def chunk_step(ci, carry0):
        pltpu.sync_copy(c_hbm.at[pl.ds(ebase + ci * CSZ, CSZ)], cbuf)
        pltpu.sync_copy(r_hbm.at[pl.ds(ebase + ci * CSZ, CSZ)], rbuf)

        def vec_step(v4, carry):
            manys = []
            for u in range(4):
                cv = cbuf[pl.ds((v4 * 4 + u) * 16, 16)]
                mu = cv == tsc[0]
                for j in range(1, 5):
                    mu = jnp.logical_or(mu, cv == tsc[j])
                manys.append(mu)
            comb = manys[0]
            for u in range(1, 4):
                comb = jnp.logical_or(comb, manys[u])
            hb = jnp.where(comb, 1, 0)
            s = hb[0]
            for q in range(1, 16):
                s = s | hb[q]

            @pl.when(s > 0)
            def _():
                for u in range(4):
                    v = v4 * 4 + u
                    cv = cbuf[pl.ds(v * 16, 16)]
                    m = [cv == tsc[j] for j in range(5)]
                    many = m[0]
                    for j in range(1, 5):
                        many = jnp.logical_or(many, m[j])
                    hbu = jnp.where(many, 1, 0)
                    su = hbu[0]
                    for q in range(1, 16):
                        su = su | hbu[q]

                    @pl.when(su > 0)
                    def _():
                        rv = rbuf[pl.ds(v * 16, 16)]
                        idxs = jnp.where(many, rv, 0)
                        pltpu.async_copy(z1t_hbm.at[idxs], zbuf, sem).wait()
                        mw = [jnp.where(m[j], 1.0, 0.0) for j in range(5)]
                        for i in range(16):
                            zr = zbuf[i]
                            w = zr[8]
                            wj = [w * mw[j][i] for j in range(5)]
                            for cc in range(8):
                                t = wbuf[5, pl.ds(cc * 16, 16)]
                                for k in range(5):
                                    t = t + zr[k] * wbuf[k, pl.ds(cc * 16, 16)]
                                x2 = jnp.maximum(t, 0.0)
                                for j in range(5):
                                    accv[j, cc] = accv[j, cc] + wj[j] * x2
            return carry

        lax.fori_loop(0, NVEC // 4, vec_step, 0)
        return carry0

    lax.fori_loop(0, NCHUNK, chunk_step, 0)
    pltpu.async_copy(z1t_hbm.at[tvec], zbuf, sem).wait()

    @pl.when(parity == 0)
    def _():
        for j in range(5):
            zr = zbuf[j]
            w = zr[8]
            for cc in range(8):
                t = wbuf[5, pl.ds(cc * 16, 16)]
                for k in range(5):
                    t = t + zr[k] * wbuf[k, pl.ds(cc * 16, 16)]
                accv[j, cc] = accv[j, cc] + w * jnp.maximum(t, 0.0)

    for j in range(5):
        dp = zbuf[j][8]
        for cc in range(8):
            accv[j, cc] = accv[j, cc] * dp
    pltpu.sync_copy(accv, out_hbm.at[wid])


_match_kernel = pl.kernel(
    _match_body,
    out_type=jax.ShapeDtypeStruct((NW, 5, 8, 16), jnp.float32),
    mesh=_mesh,
    compiler_params=pltpu.CompilerParams(use_tc_tiling_on_sc=False),
    scratch_types=[
        pltpu.VMEM((CSZ,), jnp.int32),
        pltpu.VMEM((CSZ,), jnp.int32),
        pltpu.VMEM((16, 16), jnp.float32),
        pltpu.VMEM((8, 128), jnp.float32),
        pltpu.VMEM((16,), jnp.int32),
        pltpu.VMEM((5, 8, 16), jnp.float32),
        pltpu.SemaphoreType.DMA,
    ],
)


GB = 8000
NGB = NT // GB


def _glue1_body(d0_ref, d1_ref, x_ref, y8_ref, dis_ref):
    deg = 1.0 + d0_ref[0, 0] + d1_ref[0, 0]
    dis = jax.lax.rsqrt(deg)
    x8 = jnp.concatenate(
        [x_ref[...], jnp.zeros((GB, 3), jnp.float32)], axis=1)
    y8_ref[...] = dis[:, None] * x8
    dis_ref[...] = dis[None, None, :]


def _glue1(d0, d1, x):
    return pl.pallas_call(
        _glue1_body,
        grid=(NGB,),
        in_specs=[
            pl.BlockSpec((1, 1, GB), lambda i: (i, 0, 0)),
            pl.BlockSpec((1, 1, GB), lambda i: (i, 0, 0)),
            pl.BlockSpec((GB, 5), lambda i: (i, 0)),
        ],
        out_specs=[
            pl.BlockSpec((GB, 8), lambda i: (i, 0)),
            pl.BlockSpec((1, 1, GB), lambda i: (i, 0, 0)),
        ],
        out_shape=[
            jax.ShapeDtypeStruct((NT, 8), jnp.float32),
            jax.ShapeDtypeStruct((NGB, 1, GB), jnp.float32),
        ],
    )(d0.reshape(NGB, 1, GB), d1.reshape(NGB, 1, GB), x)


def _glue2_body(s0_ref, s1_ref, y8_ref, dis_ref, z1t_ref):
    dis = dis_ref[0, 0]
    z8 = dis[:, None] * (s0_ref[...] + s1_ref[...] + y8_ref[...])
    z1t_ref[...] = jnp.concatenate(
        [z8, dis[:, None], jnp.zeros((GB, 7), jnp.float32)], axis=1)


def _glue2(s0, s1, y8, dis):
    return pl.pallas_call(
        _glue2_body,
        grid=(NGB,),
        in_specs=[
            pl.BlockSpec((GB, 8), lambda i: (i, 0)),
            pl.BlockSpec((GB, 8), lambda i: (i, 0)),
            pl.BlockSpec((GB, 8), lambda i: (i, 0)),
            pl.BlockSpec((1, 1, GB), lambda i: (i, 0, 0)),
        ],
        out_specs=pl.BlockSpec((GB, 16), lambda i: (i, 0)),
        out_shape=jax.ShapeDtypeStruct((NT, 16), jnp.float32),
    )(s0, s1, y8, dis)


def _mm(a, b):
    return jax.lax.dot_general(a, b, (((1,), (0,)), ((), ())),
                               precision=jax.lax.Precision.HIGHEST,
                               preferred_element_type=jnp.float32)


def _head_body(agg_ref, act_ref, step_ref, W2_ref, b2_ref, P1m_ref, P1a_ref,
               P1s_ref, P1b_ref, P2w_ref, P2b_ref, out_ref):
    acc = (_mm(act_ref[...], P1a_ref[...].T) + _mm(step_ref[...], P1s_ref[...].T)
           + P1b_ref[...][None, :])
    W2t = W2_ref[...].T
    b2 = b2_ref[...][None, :]
    for j in range(5):
        o2 = jnp.maximum(_mm(agg_ref[j], W2t) + b2, 0.0)
        acc = acc + _mm(o2, P1m_ref[j])
    h = jnp.maximum(acc, 0.0)
    out_ref[...] = _mm(h, P2w_ref[...].T) + P2b_ref[...][None, :]


def _head(agg, actions, steps, W2, b2, P1m, P1a, P1s, P1b, P2w, P2b):
    return pl.pallas_call(
        _head_body,
        out_shape=jax.ShapeDtypeStruct((B, EMB), jnp.float32),
    )(agg, actions, steps, W2, b2, P1m, P1a, P1s, P1b, P2w, P2b)


def kernel(graph_x, graph_edges, position_seq, available_moves, actions, steps,
           W1, b1, W2, b2, P1w, P1b, P2w, P2b):
    addition = jnp.arange(B, dtype=graph_edges.dtype) * N
    x = graph_x.reshape(-1, graph_x.shape[2])
    ge = graph_edges + addition.reshape(-1, 1, 1)
    ge = jnp.transpose(ge, (1, 0, 2)).reshape(2, -1)
    r, c = ge[0], ge[1]
    r2d = r.reshape(ROWS, 128)
    c2d = c.reshape(ROWS, 128)
    targets = jnp.concatenate(
        [available_moves + addition[:, None], (position_seq + addition)[:, None]],
        axis=1)

    degp = _deg_kernel(c2d, jnp.zeros((NT,), jnp.float32))
    y8, dis = _glue1(degp[0], degp[1], x)
    Sp = _agg_kernel(r2d, c2d, y8, jnp.zeros((NT, 8), jnp.float32))

    z1t = _glue2(Sp[0], Sp[1], y8, dis)
    tgt = jnp.repeat(targets, 2, axis=0)
    tgt = jnp.pad(tgt, ((0, 0), (0, 11)), constant_values=0)
    W1p = jnp.concatenate(
        [W1.T, b1[None, :], jnp.zeros((2, H), jnp.float32)], axis=0)
    accp = _match_kernel(c, r, z1t, tgt, W1p)
    agg = accp.reshape(B, 2, 5, H).sum(axis=1)
    agg = agg.transpose(1, 0, 2)

    P1m = P1w[:, :5 * H].reshape(EMB, 5, H).transpose(1, 2, 0)
    P1a = P1w[:, 5 * H:5 * H + 8]
    P1s = P1w[:, 5 * H + 8:]
    out = _head(agg, actions, steps, W2, b2, P1m, P1a, P1s, P1b, P2w, P2b)
    return out
